# baseline (speedup 1.0000x reference)
import math

import jax
import jax.numpy as jnp
from jax import lax
from jax.experimental import pallas as pl
from jax.experimental.pallas import tpu as pltpu
from jax.experimental.pallas import tpu_sc as plsc

VOCAB = 100000
D = 32
NF = 7
B, L = 1024, 200
N = B * L
SCALE = math.sqrt(float(D))

_info = plsc.get_sparse_core_info()
NC, NS, LANES = _info.num_cores, _info.num_subcores, _info.num_lanes
NW = NC * NS
TOK_PER_W = N // NW
ROWS_PER_W = TOK_PER_W // L
CHUNK = 40
CPR = L // CHUNK
NCHUNK = TOK_PER_W // CHUNK

_mesh = plsc.VectorSubcoreMesh(core_axis_name="c", subcore_axis_name="s")


def _body(xT_hbm, w0, w1, w2, w3, w4, w5, w6, out_hbm,
          idx_v, rows0, rows1, comb, gsem0, gsem1, osem):
    tables = (w0, w1, w2, w3, w4, w5, w6)
    wid = lax.axis_index("s") * NC + lax.axis_index("c")
    base = wid * TOK_PER_W
    brow0 = wid * ROWS_PER_W

    for f in range(NF):
        pltpu.sync_copy(xT_hbm.at[pl.ds(f * N + base, TOK_PER_W)],
                        idx_v.at[f])

    def fire7(g, rows, sem):
        c0 = jnp.minimum(g, NCHUNK - 1) * CHUNK
        for f in range(NF):
            pltpu.async_copy(tables[f].at[idx_v.at[f, pl.ds(c0, CHUNK)]],
                             rows.at[f], sem)

    def drain7(rows, sem):
        for f in range(NF):
            pltpu.make_async_copy(tables[f].at[idx_v.at[f, pl.ds(0, CHUNK)]],
                                  rows.at[f], sem).wait()

    def pack(rows):
        @pl.loop(0, CHUNK, unroll=2)
        def _tok(t):
            for f in range(NF):
                for j in range(D // LANES):
                    v = rows[f, t, pl.ds(j * LANES, LANES)]
                    comb[0, t, pl.ds(f * D + j * LANES, LANES)] = v * SCALE

    def fire_out(g):
        brow = brow0 + g // CPR
        l0 = (g % CPR) * CHUNK
        pltpu.async_copy(
            comb, out_hbm.at[pl.ds(brow, 1), pl.ds(l0, CHUNK), :], osem)

    def wait_out():
        pltpu.make_async_copy(
            comb, out_hbm.at[pl.ds(0, 1), pl.ds(0, CHUNK), :], osem).wait()

    fire7(0, rows0, gsem0)
    fire7(1, rows1, gsem1)
    drain7(rows0, gsem0)

    @pl.loop(0, NCHUNK // 2)
    def _pair(h):
        g = h * 2

        @pl.when(h > 0)
        def _():
            wait_out()
        pack(rows0)
        fire_out(g)
        fire7(g + 2, rows0, gsem0)
        drain7(rows1, gsem1)

        wait_out()
        pack(rows1)
        fire_out(g + 1)
        fire7(g + 3, rows1, gsem1)
        drain7(rows0, gsem0)

    wait_out()


_sc_call = pl.kernel(
    _body,
    out_type=jax.ShapeDtypeStruct((B, L, NF * D), jnp.float32),
    mesh=_mesh,
    scratch_types=[
        pltpu.VMEM((NF, TOK_PER_W), jnp.int32),
        pltpu.VMEM((NF, CHUNK, D), jnp.float32),
        pltpu.VMEM((NF, CHUNK, D), jnp.float32),
        pltpu.VMEM((1, CHUNK, NF * D), jnp.float32),
        pltpu.SemaphoreType.DMA,
        pltpu.SemaphoreType.DMA,
        pltpu.SemaphoreType.DMA,
    ],
    compiler_params=pltpu.CompilerParams(use_tc_tiling_on_sc=False),
)


@jax.jit
def kernel(x, W_tempo, W_chord, W_barbeat, W_type, W_pitch, W_duration,
           W_velocity):
    xT = x.reshape(N, NF).T.reshape(NF * N)
    return _sc_call(xT, W_tempo, W_chord, W_barbeat, W_type, W_pitch,
                    W_duration, W_velocity)


# speedup vs baseline: 1.0072x; 1.0072x over previous
import math

import jax
import jax.numpy as jnp
from jax import lax
from jax.experimental import pallas as pl
from jax.experimental.pallas import tpu as pltpu
from jax.experimental.pallas import tpu_sc as plsc

VOCAB = 100000
D = 32
NF = 7
B, L = 1024, 200
N = B * L
SCALE = math.sqrt(float(D))

_info = plsc.get_sparse_core_info()
NC, NS, LANES = _info.num_cores, _info.num_subcores, _info.num_lanes
NW = NC * NS
TOK_PER_W = N // NW
CHUNK = 128
NCHUNK = TOK_PER_W // CHUNK

_mesh = plsc.VectorSubcoreMesh(core_axis_name="c", subcore_axis_name="s")


def _body(xT_hbm, w0, w1, w2, w3, w4, w5, w6, out_hbm,
          idx_v, rows0, rows1, comb, gsem0, gsem1, osem):
    tables = (w0, w1, w2, w3, w4, w5, w6)
    wid = lax.axis_index("s") * NC + lax.axis_index("c")
    base = wid * TOK_PER_W

    for f in range(NF):
        pltpu.sync_copy(xT_hbm.at[pl.ds(f * N + base, TOK_PER_W)],
                        idx_v.at[f])

    HALF = CHUNK // 2

    def fire7(g, rows, sem):
        c0 = jnp.minimum(g, NCHUNK - 1) * CHUNK
        for f in range(NF):
            for h in (0, HALF):
                pltpu.async_copy(
                    tables[f].at[idx_v.at[f, pl.ds(c0 + h, HALF)]],
                    rows.at[f, pl.ds(h, HALF)], sem)

    def drain7(rows, sem):
        for f in range(NF):
            for h in (0, HALF):
                pltpu.make_async_copy(
                    tables[f].at[idx_v.at[f, pl.ds(0, HALF)]],
                    rows.at[f, pl.ds(h, HALF)], sem).wait()

    def pack(rows):
        @pl.loop(0, CHUNK, unroll=2)
        def _tok(t):
            for f in range(NF):
                for j in range(D // LANES):
                    v = rows[f, t, pl.ds(j * LANES, LANES)]
                    comb[t, pl.ds(f * D + j * LANES, LANES)] = v * SCALE

    def fire_out(g):
        tok0 = base + g * CHUNK
        pltpu.async_copy(comb, out_hbm.at[pl.ds(tok0, CHUNK), :], osem)

    def wait_out():
        pltpu.make_async_copy(comb, out_hbm.at[pl.ds(0, CHUNK), :],
                              osem).wait()

    fire7(0, rows0, gsem0)
    fire7(1, rows1, gsem1)
    drain7(rows0, gsem0)

    @pl.loop(0, NCHUNK // 2)
    def _pair(h):
        g = h * 2

        @pl.when(h > 0)
        def _():
            wait_out()
        pack(rows0)
        fire_out(g)
        fire7(g + 2, rows0, gsem0)
        drain7(rows1, gsem1)

        wait_out()
        pack(rows1)
        fire_out(g + 1)
        fire7(g + 3, rows1, gsem1)
        drain7(rows0, gsem0)

    wait_out()


_sc_call = pl.kernel(
    _body,
    out_type=jax.ShapeDtypeStruct((N, NF * D), jnp.float32),
    mesh=_mesh,
    scratch_types=[
        pltpu.VMEM((NF, TOK_PER_W), jnp.int32),
        pltpu.VMEM((NF, CHUNK, D), jnp.float32),
        pltpu.VMEM((NF, CHUNK, D), jnp.float32),
        pltpu.VMEM((CHUNK, NF * D), jnp.float32),
        pltpu.SemaphoreType.DMA,
        pltpu.SemaphoreType.DMA,
        pltpu.SemaphoreType.DMA,
    ],
    compiler_params=pltpu.CompilerParams(use_tc_tiling_on_sc=False),
)


@jax.jit
def kernel(x, W_tempo, W_chord, W_barbeat, W_type, W_pitch, W_duration,
           W_velocity):
    xT = x.reshape(N, NF).T.reshape(NF * N)
    out = _sc_call(xT, W_tempo, W_chord, W_barbeat, W_type, W_pitch,
                   W_duration, W_velocity)
    return out.reshape(B, L, NF * D)
